# baseline (speedup 1.0000x reference)
import functools

import jax
import jax.numpy as jnp
from jax import lax
from jax.experimental import pallas as pl
from jax.experimental.pallas import tpu as pltpu
from jax.experimental.pallas import tpu_sc as plsc

D = 64
BATCH = 4096
SEQ = 200
N = BATCH * SEQ
NC = 2
NS = 16
NW = NC * NS
PER_W = N // NW
C = 256
CHUNKS = PER_W // C
SCALE = 8.0


def _scale_chunk(src, dst):
    def row(r, carry):
        for c in range(D // 16):
            sl = pl.ds(16 * c, 16)
            dst[r, sl] = src[r, sl] * SCALE
        return carry

    lax.fori_loop(0, C, row, 0, unroll=2)


def _body(table_hbm, idx_hbm, out_hbm, idx_v, g0, g1, s0, s1,
          gsem0, gsem1, ssem0, ssem1):
    wid = lax.axis_index("s") * NC + lax.axis_index("c")
    base = wid * PER_W

    pltpu.sync_copy(idx_hbm.at[pl.ds(base, PER_W)], idx_v)

    def start_gather(g, rows, sem):
        pltpu.async_copy(table_hbm.at[idx_v.at[pl.ds(g * C, C)]], rows, sem)

    def start_scatter(g, rows, sem):
        pltpu.async_copy(rows, out_hbm.at[pl.ds(base + g * C, C)], sem)

    start_gather(0, g0, gsem0)
    start_gather(1, g1, gsem1)

    def pair(t, carry):
        ga = 2 * t
        gb = ga + 1

        @pl.when(t > 0)
        def _():
            pltpu.make_async_copy(s0, out_hbm.at[pl.ds(0, C)], ssem0).wait()

        pltpu.make_async_copy(table_hbm.at[idx_v.at[pl.ds(0, C)]], g0,
                              gsem0).wait()
        _scale_chunk(g0, s0)

        @pl.when(ga + 2 < CHUNKS)
        def _():
            start_gather(ga + 2, g0, gsem0)

        start_scatter(ga, s0, ssem0)

        @pl.when(t > 0)
        def _():
            pltpu.make_async_copy(s1, out_hbm.at[pl.ds(0, C)], ssem1).wait()

        pltpu.make_async_copy(table_hbm.at[idx_v.at[pl.ds(0, C)]], g1,
                              gsem1).wait()
        _scale_chunk(g1, s1)

        @pl.when(gb + 2 < CHUNKS)
        def _():
            start_gather(gb + 2, g1, gsem1)

        start_scatter(gb, s1, ssem1)
        return carry

    lax.fori_loop(0, CHUNKS // 2, pair, 0)

    pltpu.make_async_copy(s0, out_hbm.at[pl.ds(0, C)], ssem0).wait()
    pltpu.make_async_copy(s1, out_hbm.at[pl.ds(0, C)], ssem1).wait()


def kernel(input_ids, W):
    idx = input_ids.reshape(N).astype(jnp.int32)
    mesh = plsc.VectorSubcoreMesh(core_axis_name="c", subcore_axis_name="s")
    f = functools.partial(
        pl.kernel,
        mesh=mesh,
        compiler_params=pltpu.CompilerParams(use_tc_tiling_on_sc=False),
        out_type=jax.ShapeDtypeStruct((N, D), jnp.float32),
        scratch_types=[
            pltpu.VMEM((PER_W,), jnp.int32),
            pltpu.VMEM((C, D), jnp.float32),
            pltpu.VMEM((C, D), jnp.float32),
            pltpu.VMEM((C, D), jnp.float32),
            pltpu.VMEM((C, D), jnp.float32),
            pltpu.SemaphoreType.DMA,
            pltpu.SemaphoreType.DMA,
            pltpu.SemaphoreType.DMA,
            pltpu.SemaphoreType.DMA,
        ],
    )(_body)
    out = f(W, idx)
    return out.reshape(BATCH, SEQ, D)
